# baseline (speedup 1.0000x reference)
import dataclasses
import functools

import jax
import jax.numpy as jnp
from jax import lax
from jax.experimental import pallas as pl
from jax.experimental.pallas import tpu as pltpu
from jax.experimental.pallas import tpu_sc as plsc

B, L, D, MAX_LEN = 4, 2048, 1024, 2048
N = B * L
NUM_WORKERS = 32
ROWS_PER_WORKER = N // NUM_WORKERS
CHUNK = 32
LANES = 16
UNROLL = 8

_CP = pltpu.CompilerParams()
if "needs_layout_passes" in pltpu.CompilerParams.__dataclass_fields__:
    _CP = dataclasses.replace(_CP, needs_layout_passes=False)


@functools.partial(
    pl.kernel,
    out_type=jax.ShapeDtypeStruct((N * D,), jnp.float32),
    mesh=plsc.VectorSubcoreMesh(core_axis_name="c", subcore_axis_name="s"),
    compiler_params=_CP,
    scratch_types=[
        pltpu.VMEM((CHUNK,), jnp.int32),
        pltpu.VMEM((CHUNK,), jnp.float32),
        pltpu.VMEM((CHUNK * D,), jnp.float32),
        pltpu.VMEM((CHUNK, D), jnp.float32),
        pltpu.SemaphoreType.DMA,
        pltpu.SemaphoreType.DMA,
    ],
)
def _sc_masked_pe(x_hbm, mask_hbm, pos_hbm, pe_hbm, out_hbm,
                  idx_v, msk_v, x_v, pe_v, sem_a, sem_b):
    wid = lax.axis_index("s") * 2 + lax.axis_index("c")
    base = wid * ROWS_PER_WORKER

    @pl.loop(0, ROWS_PER_WORKER, step=CHUNK)
    def _chunk(off):
        rb = base + off
        pltpu.sync_copy(pos_hbm.at[pl.ds(rb, CHUNK)], idx_v)
        pltpu.sync_copy(mask_hbm.at[pl.ds(rb, CHUNK)], msk_v)
        gat = pltpu.async_copy(pe_hbm.at[idx_v], pe_v, sem_a)
        lin = pltpu.async_copy(x_hbm.at[pl.ds(rb * D, CHUNK * D)], x_v, sem_b)
        gat.wait()
        lin.wait()

        @pl.loop(0, CHUNK)
        def _row(r):
            m = plsc.load_gather(msk_v, [jnp.full((LANES,), r, jnp.int32)])
            rbase = r * D

            @pl.loop(0, D, step=LANES * UNROLL)
            def _col(j):
                for k in range(UNROLL):
                    xs = pl.ds(rbase + j + k * LANES, LANES)
                    x_v[xs] = x_v[xs] + m * pe_v[r, pl.ds(j + k * LANES, LANES)]

        pltpu.sync_copy(x_v, out_hbm.at[pl.ds(rb * D, CHUNK * D)])


@jax.jit
def kernel(x, source_mask, positions, positional_encoding):
    x2 = x.reshape(N * D)
    mask = source_mask.reshape(N).astype(jnp.float32)
    pos = positions.reshape(N).astype(jnp.int32)
    out = _sc_masked_pe(x2, mask, pos, positional_encoding)
    return out.reshape(B, L, D)


# speedup vs baseline: 1.5589x; 1.5589x over previous
import dataclasses
import functools

import jax
import jax.numpy as jnp
from jax import lax
from jax.experimental import pallas as pl
from jax.experimental.pallas import tpu as pltpu
from jax.experimental.pallas import tpu_sc as plsc

B, L, D, MAX_LEN = 4, 2048, 1024, 2048
N = B * L
NUM_WORKERS = 32
ROWS_PER_WORKER = N // NUM_WORKERS
CHUNK = 32
LANES = 16
UNROLL = 8

_CP = pltpu.CompilerParams()
if "needs_layout_passes" in pltpu.CompilerParams.__dataclass_fields__:
    _CP = dataclasses.replace(_CP, needs_layout_passes=False)


@functools.partial(
    pl.kernel,
    out_type=jax.ShapeDtypeStruct((N * D,), jnp.float32),
    mesh=plsc.VectorSubcoreMesh(core_axis_name="c", subcore_axis_name="s"),
    compiler_params=_CP,
    scratch_types=[
        pltpu.VMEM((CHUNK,), jnp.int32),
        pltpu.VMEM((CHUNK,), jnp.float32),
        pltpu.VMEM((CHUNK * D,), jnp.float32),
        pltpu.VMEM((CHUNK, D), jnp.float32),
        pltpu.SemaphoreType.DMA,
        pltpu.SemaphoreType.DMA,
    ],
)
def _sc_masked_pe(x_hbm, mask_hbm, pos_hbm, pe_hbm, out_hbm,
                  idx_v, msk_v, x_v, pe_v, sem_a, sem_b):
    wid = lax.axis_index("s") * 2 + lax.axis_index("c")
    base = wid * ROWS_PER_WORKER

    @pl.loop(0, ROWS_PER_WORKER, step=CHUNK)
    def _chunk(off):
        rb = base + off
        pltpu.sync_copy(pos_hbm.at[pl.ds(rb, CHUNK)], idx_v)
        pltpu.sync_copy(mask_hbm.at[pl.ds(rb, CHUNK)], msk_v)
        gat = pltpu.async_copy(pe_hbm.at[idx_v], pe_v, sem_a)
        lin = pltpu.async_copy(x_hbm.at[pl.ds(rb * D, CHUNK * D)], x_v, sem_b)
        gat.wait()
        lin.wait()

        @plsc.parallel_loop(0, CHUNK)
        def _row(r):
            m = plsc.load_gather(msk_v, [jnp.full((LANES,), r, jnp.int32)])
            rbase = r * D

            @plsc.parallel_loop(0, D, LANES, unroll=UNROLL)
            def _col(j):
                xs = pl.ds(rbase + j, LANES)
                x_v[xs] = x_v[xs] + m * pe_v[r, pl.ds(j, LANES)]

        pltpu.sync_copy(x_v, out_hbm.at[pl.ds(rb * D, CHUNK * D)])


@jax.jit
def kernel(x, source_mask, positions, positional_encoding):
    x2 = x.reshape(N * D)
    mask = source_mask.reshape(N).astype(jnp.float32)
    pos = positions.reshape(N).astype(jnp.int32)
    out = _sc_masked_pe(x2, mask, pos, positional_encoding)
    return out.reshape(B, L, D)


# speedup vs baseline: 1.9091x; 1.2246x over previous
import dataclasses
import functools

import jax
import jax.numpy as jnp
from jax import lax
from jax.experimental import pallas as pl
from jax.experimental.pallas import tpu as pltpu
from jax.experimental.pallas import tpu_sc as plsc

B, L, D, MAX_LEN = 4, 2048, 1024, 2048
N = B * L
NUM_WORKERS = 32
ROWS_PER_WORKER = N // NUM_WORKERS
CHUNK = 16
NCHUNK = ROWS_PER_WORKER // CHUNK
LANES = 16
UNROLL = 8

_CP = pltpu.CompilerParams()
if "needs_layout_passes" in pltpu.CompilerParams.__dataclass_fields__:
    _CP = dataclasses.replace(_CP, needs_layout_passes=False)


@functools.partial(
    pl.kernel,
    out_type=jax.ShapeDtypeStruct((N * D,), jnp.float32),
    mesh=plsc.VectorSubcoreMesh(core_axis_name="c", subcore_axis_name="s"),
    compiler_params=_CP,
    scratch_types=[
        pltpu.VMEM((ROWS_PER_WORKER,), jnp.int32),
        pltpu.VMEM((ROWS_PER_WORKER,), jnp.float32),
        pltpu.VMEM((CHUNK * D,), jnp.float32),
        pltpu.VMEM((CHUNK * D,), jnp.float32),
        pltpu.VMEM((CHUNK, D), jnp.float32),
        pltpu.VMEM((CHUNK, D), jnp.float32),
        pltpu.SemaphoreType.DMA,
        pltpu.SemaphoreType.DMA,
        pltpu.SemaphoreType.DMA,
        pltpu.SemaphoreType.DMA,
        pltpu.SemaphoreType.DMA,
        pltpu.SemaphoreType.DMA,
    ],
)
def _sc_masked_pe(x_hbm, mask_hbm, pos_hbm, pe_hbm, out_hbm,
                  pos_v, msk_v, x0, x1, pe0, pe1,
                  sg0, sg1, sx0, sx1, ss0, ss1):
    wid = lax.axis_index("s") * 2 + lax.axis_index("c")
    base = wid * ROWS_PER_WORKER

    pltpu.sync_copy(pos_hbm.at[pl.ds(base, ROWS_PER_WORKER)], pos_v)
    pltpu.sync_copy(mask_hbm.at[pl.ds(base, ROWS_PER_WORKER)], msk_v)

    bufs = ((x0, pe0, sg0, sx0, ss0), (x1, pe1, sg1, sx1, ss1))

    def issue(c, buf):
        x_b, pe_b, sg, sx, _ = buf
        off = pl.multiple_of(c * CHUNK, CHUNK)
        pltpu.async_copy(pe_hbm.at[pos_v.at[pl.ds(off, CHUNK)]], pe_b, sg)
        pltpu.async_copy(
            x_hbm.at[pl.ds(pl.multiple_of((base + off) * D, CHUNK * D),
                           CHUNK * D)], x_b, sx)

    issue(0, bufs[0])
    issue(1, bufs[1])

    @pl.loop(0, NCHUNK, step=2)
    def _pair(i):
        for b in range(2):
            c = i + b
            x_b, pe_b, sg, sx, ss = bufs[b]
            off = pl.multiple_of(c * CHUNK, CHUNK)
            hoff = pl.multiple_of((base + off) * D, CHUNK * D)
            pltpu.make_async_copy(pe_hbm.at[pl.ds(0, CHUNK)], pe_b, sg).wait()
            pltpu.make_async_copy(x_hbm.at[pl.ds(0, CHUNK * D)], x_b, sx).wait()

            @plsc.parallel_loop(0, CHUNK)
            def _row(r):
                m = plsc.load_gather(
                    msk_v, [jnp.full((LANES,), off + r, jnp.int32)])
                rbase = r * D

                @plsc.parallel_loop(0, D, LANES, unroll=UNROLL)
                def _col(j):
                    xs = pl.ds(rbase + j, LANES)
                    x_b[xs] = x_b[xs] + m * pe_b[r, pl.ds(j, LANES)]

            pltpu.async_copy(x_b, out_hbm.at[pl.ds(hoff, CHUNK * D)], ss)

            @pl.when(c + 2 < NCHUNK)
            def _refill():
                pltpu.make_async_copy(
                    x_b, out_hbm.at[pl.ds(0, CHUNK * D)], ss).wait()
                issue(c + 2, bufs[b])

    pltpu.make_async_copy(x0, out_hbm.at[pl.ds(0, CHUNK * D)], ss0).wait()
    pltpu.make_async_copy(x1, out_hbm.at[pl.ds(0, CHUNK * D)], ss1).wait()


@jax.jit
def kernel(x, source_mask, positions, positional_encoding):
    x2 = x.reshape(N * D)
    mask = source_mask.reshape(N).astype(jnp.float32)
    pos = positions.reshape(N).astype(jnp.int32)
    out = _sc_masked_pe(x2, mask, pos, positional_encoding)
    return out.reshape(B, L, D)


# speedup vs baseline: 3.9080x; 2.0470x over previous
import dataclasses
import functools

import jax
import jax.numpy as jnp
from jax import lax
from jax.experimental import pallas as pl
from jax.experimental.pallas import tpu as pltpu
from jax.experimental.pallas import tpu_sc as plsc

B, L, D, MAX_LEN = 4, 2048, 1024, 2048
N = B * L
NUM_WORKERS = 32
ROWS_PER_WORKER = N // NUM_WORKERS
CHUNK = 16
NCHUNK = ROWS_PER_WORKER // CHUNK
LANES = 16
UNROLL = 8
CT = D // 128
NSEG = CHUNK * CT

_CP = pltpu.CompilerParams()
if "needs_layout_passes" in pltpu.CompilerParams.__dataclass_fields__:
    _CP = dataclasses.replace(_CP, needs_layout_passes=False)


@functools.partial(
    pl.kernel,
    out_type=jax.ShapeDtypeStruct((N * D,), jnp.float32),
    mesh=plsc.VectorSubcoreMesh(core_axis_name="c", subcore_axis_name="s"),
    compiler_params=_CP,
    scratch_types=[
        pltpu.VMEM((ROWS_PER_WORKER,), jnp.int32),
        pltpu.VMEM((ROWS_PER_WORKER,), jnp.float32),
        pltpu.VMEM((CHUNK * D,), jnp.float32),
        pltpu.VMEM((CHUNK * D,), jnp.float32),
        pltpu.VMEM((NSEG, 128), jnp.float32),
        pltpu.VMEM((NSEG, 128), jnp.float32),
        pltpu.VMEM((NSEG,), jnp.int32),
        pltpu.VMEM((NSEG,), jnp.int32),
        pltpu.SemaphoreType.DMA,
        pltpu.SemaphoreType.DMA,
        pltpu.SemaphoreType.DMA,
        pltpu.SemaphoreType.DMA,
        pltpu.SemaphoreType.DMA,
        pltpu.SemaphoreType.DMA,
    ],
)
def _sc_masked_pe(x_hbm, mask_hbm, pos_hbm, pe_hbm, out_hbm,
                  pos_v, msk_v, x0, x1, pe0, pe1, ix0, ix1,
                  sg0, sg1, sx0, sx1, ss0, ss1):
    wid = lax.axis_index("s") * 2 + lax.axis_index("c")
    base = wid * ROWS_PER_WORKER

    pltpu.sync_copy(pos_hbm.at[pl.ds(base, ROWS_PER_WORKER)], pos_v)
    pltpu.sync_copy(mask_hbm.at[pl.ds(base, ROWS_PER_WORKER)], msk_v)

    it = lax.iota(jnp.int32, LANES)
    dst0 = ((it >> 3) << 6) + (it & 7)

    bufs = ((x0, pe0, ix0, sg0, sx0, ss0), (x1, pe1, ix1, sg1, sx1, ss1))

    def issue(c, buf):
        x_b, pe_b, ix_b, sg, sx, _ = buf
        off = pl.multiple_of(c * CHUNK, CHUNK)
        p = plsc.load_gather(pos_v, [off + it])
        seg0 = ((p >> 3) << 6) + (p & 7)
        for ct in range(CT):
            plsc.store_scatter(ix_b, [dst0 + (ct << 3)], seg0 + (ct << 3))
        pltpu.async_copy(pe_hbm.at[ix_b], pe_b, sg)
        pltpu.async_copy(
            x_hbm.at[pl.ds(pl.multiple_of((base + off) * D, CHUNK * D),
                           CHUNK * D)], x_b, sx)

    issue(0, bufs[0])
    issue(1, bufs[1])

    @pl.loop(0, NCHUNK, step=2)
    def _pair(i):
        for b in range(2):
            c = i + b
            x_b, pe_b, ix_b, sg, sx, ss = bufs[b]
            off = pl.multiple_of(c * CHUNK, CHUNK)
            hoff = pl.multiple_of((base + off) * D, CHUNK * D)
            pltpu.make_async_copy(pe_hbm.at[pl.ds(0, NSEG)], pe_b, sg).wait()
            pltpu.make_async_copy(x_hbm.at[pl.ds(0, CHUNK * D)], x_b, sx).wait()

            @plsc.parallel_loop(0, CHUNK)
            def _row(r):
                m = plsc.load_gather(
                    msk_v, [jnp.full((LANES,), off + r, jnp.int32)])
                rbase = ((r >> 3) << 13) + ((r & 7) << 7)
                prow0 = ((r >> 3) << 6) + (r & 7)

                @plsc.parallel_loop(0, CT)
                def _ctile(ct):
                    xtb = rbase + (ct << 10)
                    prow = prow0 + (ct << 3)

                    @plsc.parallel_loop(0, 128, LANES, unroll=UNROLL)
                    def _col(j):
                        xs = pl.ds(xtb + j, LANES)
                        x_b[xs] = x_b[xs] + m * pe_b[prow, pl.ds(j, LANES)]

            pltpu.async_copy(x_b, out_hbm.at[pl.ds(hoff, CHUNK * D)], ss)

            @pl.when(c + 2 < NCHUNK)
            def _refill():
                pltpu.make_async_copy(
                    x_b, out_hbm.at[pl.ds(0, CHUNK * D)], ss).wait()
                issue(c + 2, bufs[b])

    pltpu.make_async_copy(x0, out_hbm.at[pl.ds(0, CHUNK * D)], ss0).wait()
    pltpu.make_async_copy(x1, out_hbm.at[pl.ds(0, CHUNK * D)], ss1).wait()


@jax.jit
def kernel(x, source_mask, positions, positional_encoding):
    xb = jnp.transpose(x.reshape(N // 8, 8, CT, 128), (0, 2, 1, 3)).reshape(N * D)
    peb = jnp.transpose(
        positional_encoding.reshape(MAX_LEN // 8, 8, CT, 128),
        (0, 2, 1, 3)).reshape(MAX_LEN * CT, 128)
    mask = source_mask.reshape(N).astype(jnp.float32)
    pos = positions.reshape(N).astype(jnp.int32)
    out = _sc_masked_pe(xb, mask, pos, peb)
    return jnp.transpose(
        out.reshape(N // 8, CT, 8, 128), (0, 2, 1, 3)).reshape(B, L, D)


# speedup vs baseline: 4.0798x; 1.0440x over previous
import dataclasses
import functools

import jax
import jax.numpy as jnp
from jax import lax
from jax.experimental import pallas as pl
from jax.experimental.pallas import tpu as pltpu
from jax.experimental.pallas import tpu_sc as plsc

B, L, D, MAX_LEN = 4, 2048, 1024, 2048
N = B * L
NUM_WORKERS = 32
ROWS_PER_WORKER = N // NUM_WORKERS
CHUNK = 8
NCHUNK = ROWS_PER_WORKER // CHUNK
RING = 4
LANES = 16
UNROLL = 8
CT = D // 128
NSEG = CHUNK * CT

_CP = pltpu.CompilerParams()
if "needs_layout_passes" in pltpu.CompilerParams.__dataclass_fields__:
    _CP = dataclasses.replace(_CP, needs_layout_passes=False)


@functools.partial(
    pl.kernel,
    out_type=jax.ShapeDtypeStruct((N * D,), jnp.float32),
    mesh=plsc.VectorSubcoreMesh(core_axis_name="c", subcore_axis_name="s"),
    compiler_params=_CP,
    scratch_types=(
        [pltpu.VMEM((ROWS_PER_WORKER,), jnp.int32),
         pltpu.VMEM((ROWS_PER_WORKER,), jnp.float32)]
        + [pltpu.VMEM((CHUNK * D,), jnp.float32)] * RING
        + [pltpu.VMEM((NSEG, 128), jnp.float32)] * RING
        + [pltpu.VMEM((NSEG,), jnp.int32)] * RING
        + [pltpu.SemaphoreType.DMA] * (3 * RING)
    ),
)
def _sc_masked_pe(x_hbm, mask_hbm, pos_hbm, pe_hbm, out_hbm,
                  pos_v, msk_v, *scratch):
    xb_ = scratch[0:RING]
    pb_ = scratch[RING:2 * RING]
    ib_ = scratch[2 * RING:3 * RING]
    sg_ = scratch[3 * RING:4 * RING]
    sx_ = scratch[4 * RING:5 * RING]
    ss_ = scratch[5 * RING:6 * RING]
    bufs = tuple(zip(xb_, pb_, ib_, sg_, sx_, ss_))

    wid = lax.axis_index("s") * 2 + lax.axis_index("c")
    base = wid * ROWS_PER_WORKER

    pltpu.sync_copy(pos_hbm.at[pl.ds(base, ROWS_PER_WORKER)], pos_v)
    pltpu.sync_copy(mask_hbm.at[pl.ds(base, ROWS_PER_WORKER)], msk_v)

    it = lax.iota(jnp.int32, LANES)
    dst0 = ((it >> 3) << 3) + (it & 7)

    def issue(c, buf):
        x_b, pe_b, ix_b, sg, sx, _ = buf
        off = pl.multiple_of(c * CHUNK, CHUNK)
        p = plsc.load_gather(pos_v, [off + (it & 7)])
        seg0 = ((p >> 3) << 6) + (p & 7) + ((it >> 3) << 3)
        for h in range(CT // 2):
            plsc.store_scatter(ix_b, [dst0 + (h << 4)], seg0 + (h << 4))
        pltpu.async_copy(pe_hbm.at[ix_b], pe_b, sg)
        pltpu.async_copy(
            x_hbm.at[pl.ds(pl.multiple_of((base + off) * D, CHUNK * D),
                           CHUNK * D)], x_b, sx)

    for b in range(RING):
        issue(b, bufs[b])

    @pl.loop(0, NCHUNK, step=RING)
    def _ring(i):
        for b in range(RING):
            c = i + b
            x_b, pe_b, ix_b, sg, sx, ss = bufs[b]
            off = pl.multiple_of(c * CHUNK, CHUNK)
            hoff = pl.multiple_of((base + off) * D, CHUNK * D)
            pltpu.make_async_copy(pe_hbm.at[pl.ds(0, NSEG)], pe_b, sg).wait()
            pltpu.make_async_copy(x_hbm.at[pl.ds(0, CHUNK * D)], x_b, sx).wait()

            @plsc.parallel_loop(0, CHUNK)
            def _row(r):
                m = plsc.load_gather(
                    msk_v, [jnp.full((LANES,), off + r, jnp.int32)])
                rbase = (r & 7) << 7
                prow0 = r & 7

                @plsc.parallel_loop(0, CT)
                def _ctile(ct):
                    xtb = rbase + (ct << 10)
                    prow = prow0 + (ct << 3)

                    @plsc.parallel_loop(0, 128, LANES, unroll=UNROLL)
                    def _col(j):
                        xs = pl.ds(xtb + j, LANES)
                        x_b[xs] = x_b[xs] + m * pe_b[prow, pl.ds(j, LANES)]

            pltpu.async_copy(x_b, out_hbm.at[pl.ds(hoff, CHUNK * D)], ss)

            @pl.when(c + RING < NCHUNK)
            def _refill():
                pltpu.make_async_copy(
                    x_b, out_hbm.at[pl.ds(0, CHUNK * D)], ss).wait()
                issue(c + RING, bufs[b])

    for b in range(RING):
        pltpu.make_async_copy(
            bufs[b][0], out_hbm.at[pl.ds(0, CHUNK * D)], bufs[b][5]).wait()


@jax.jit
def kernel(x, source_mask, positions, positional_encoding):
    xb = jnp.transpose(x.reshape(N // 8, 8, CT, 128), (0, 2, 1, 3)).reshape(N * D)
    peb = jnp.transpose(
        positional_encoding.reshape(MAX_LEN // 8, 8, CT, 128),
        (0, 2, 1, 3)).reshape(MAX_LEN * CT, 128)
    mask = source_mask.reshape(N).astype(jnp.float32)
    pos = positions.reshape(N).astype(jnp.int32)
    out = _sc_masked_pe(xb, mask, pos, peb)
    return jnp.transpose(
        out.reshape(N // 8, CT, 8, 128), (0, 2, 1, 3)).reshape(B, L, D)


# speedup vs baseline: 4.1186x; 1.0095x over previous
import dataclasses
import functools

import jax
import jax.numpy as jnp
from jax import lax
from jax.experimental import pallas as pl
from jax.experimental.pallas import tpu as pltpu
from jax.experimental.pallas import tpu_sc as plsc

B, L, D, MAX_LEN = 4, 2048, 1024, 2048
N = B * L
NUM_WORKERS = 32
ROWS_PER_WORKER = N // NUM_WORKERS
CHUNK = 8
NCHUNK = ROWS_PER_WORKER // CHUNK
RING = 4
LANES = 16
UNROLL = 8
CT = D // 128
NSEG = CHUNK * CT

_CP = pltpu.CompilerParams()
if "needs_layout_passes" in pltpu.CompilerParams.__dataclass_fields__:
    _CP = dataclasses.replace(_CP, needs_layout_passes=False)


@functools.partial(
    pl.kernel,
    out_type=jax.ShapeDtypeStruct((N * D,), jnp.float32),
    mesh=plsc.VectorSubcoreMesh(core_axis_name="c", subcore_axis_name="s"),
    compiler_params=_CP,
    scratch_types=(
        [pltpu.VMEM((ROWS_PER_WORKER,), jnp.int32),
         pltpu.VMEM((ROWS_PER_WORKER,), jnp.float32)]
        + [pltpu.VMEM((CHUNK * D,), jnp.float32)] * RING
        + [pltpu.VMEM((NSEG, 128), jnp.float32)] * RING
        + [pltpu.VMEM((NSEG,), jnp.int32)] * RING
        + [pltpu.SemaphoreType.DMA] * (3 * RING + 1)
    ),
)
def _sc_masked_pe(x_hbm, mask_hbm, pos_hbm, pe_hbm, out_hbm,
                  pos_v, msk_v, *scratch):
    xb_ = scratch[0:RING]
    pb_ = scratch[RING:2 * RING]
    ib_ = scratch[2 * RING:3 * RING]
    sg_ = scratch[3 * RING:4 * RING]
    sx_ = scratch[4 * RING:5 * RING]
    ss_ = scratch[5 * RING:6 * RING]
    st_sem = scratch[6 * RING]
    bufs = tuple(zip(xb_, pb_, ib_, sg_, sx_, ss_))

    wid = lax.axis_index("s") * 2 + lax.axis_index("c")
    base = wid * ROWS_PER_WORKER

    cp_pos = pltpu.async_copy(
        pos_hbm.at[pl.ds(base, ROWS_PER_WORKER)], pos_v, st_sem)
    cp_msk = pltpu.async_copy(
        mask_hbm.at[pl.ds(base, ROWS_PER_WORKER)], msk_v, st_sem)
    cp_pos.wait()
    cp_msk.wait()

    it = lax.iota(jnp.int32, LANES)
    dst0 = ((it >> 3) << 3) + (it & 7)

    def issue(c, buf):
        x_b, pe_b, ix_b, sg, sx, _ = buf
        off = pl.multiple_of(c * CHUNK, CHUNK)
        p = plsc.load_gather(pos_v, [off + (it & 7)])
        seg0 = ((p >> 3) << 6) + (p & 7) + ((it >> 3) << 3)
        for h in range(CT // 2):
            plsc.store_scatter(ix_b, [dst0 + (h << 4)], seg0 + (h << 4))
        pltpu.async_copy(pe_hbm.at[ix_b], pe_b, sg)
        pltpu.async_copy(
            x_hbm.at[pl.ds(pl.multiple_of((base + off) * D, CHUNK * D),
                           CHUNK * D)], x_b, sx)

    for b in range(RING):
        issue(b, bufs[b])

    @pl.loop(0, NCHUNK, step=RING)
    def _ring(i):
        for b in range(RING):
            c = i + b
            x_b, pe_b, ix_b, sg, sx, ss = bufs[b]
            off = pl.multiple_of(c * CHUNK, CHUNK)
            hoff = pl.multiple_of((base + off) * D, CHUNK * D)
            pltpu.make_async_copy(pe_hbm.at[pl.ds(0, NSEG)], pe_b, sg).wait()
            pltpu.make_async_copy(x_hbm.at[pl.ds(0, CHUNK * D)], x_b, sx).wait()

            @plsc.parallel_loop(0, CHUNK)
            def _row(r):
                m = plsc.load_gather(
                    msk_v, [jnp.full((LANES,), off + r, jnp.int32)])
                rbase = (r & 7) << 7
                prow0 = r & 7

                @plsc.parallel_loop(0, CT)
                def _ctile(ct):
                    xtb = rbase + (ct << 10)
                    prow = prow0 + (ct << 3)

                    @plsc.parallel_loop(0, 128, LANES, unroll=UNROLL)
                    def _col(j):
                        xs = pl.ds(xtb + j, LANES)
                        x_b[xs] = x_b[xs] + m * pe_b[prow, pl.ds(j, LANES)]

            pltpu.async_copy(x_b, out_hbm.at[pl.ds(hoff, CHUNK * D)], ss)

            @pl.when(c + RING < NCHUNK)
            def _refill():
                pltpu.make_async_copy(
                    x_b, out_hbm.at[pl.ds(0, CHUNK * D)], ss).wait()
                issue(c + RING, bufs[b])

    for b in range(RING):
        pltpu.make_async_copy(
            bufs[b][0], out_hbm.at[pl.ds(0, CHUNK * D)], bufs[b][5]).wait()


@jax.jit
def kernel(x, source_mask, positions, positional_encoding):
    xb = jnp.transpose(x.reshape(N // 8, 8, CT, 128), (0, 2, 1, 3)).reshape(N * D)
    peb = jnp.transpose(
        positional_encoding.reshape(MAX_LEN // 8, 8, CT, 128),
        (0, 2, 1, 3)).reshape(MAX_LEN * CT, 128)
    mask = source_mask.reshape(N).astype(jnp.float32)
    pos = positions.reshape(N).astype(jnp.int32)
    out = _sc_masked_pe(xb, mask, pos, peb)
    return jnp.transpose(
        out.reshape(N // 8, CT, 8, 128), (0, 2, 1, 3)).reshape(B, L, D)
